# baseline (speedup 1.0000x reference)
import functools

import jax
import jax.numpy as jnp
from jax import lax
from jax.experimental import pallas as pl
from jax.experimental.pallas import tpu as pltpu
from jax.experimental.pallas import tpu_sc as plsc

_N = 16
_J = 128
_SEG = 1024
_K = 20

_NC, _NS = 2, 16
_NW = _NC * _NS
_SEGS_PER_W = (_N * _J) // _NW
_ELEMS_PER_W = _SEGS_PER_W * _SEG
_GROUPS = _SEGS_PER_W // 16
_CHUNK = 16 * _SEG

_DN = (((1,), (1,)), ((), ()))


def _sc_pool_body(x_hbm, out_hbm, buf, acc_ref, s0, s1, s2, s3):
    wid = lax.axis_index("s") * _NC + lax.axis_index("c")
    row = wid // 2
    half = wid % 2
    col = half * _ELEMS_PER_W
    sems = (s0, s1, s2, s3)

    handles = [
        pltpu.async_copy(x_hbm.at[row, pl.ds(col + g * _CHUNK, _CHUNK)],
                         buf.at[pl.ds(g * _CHUNK, _CHUNK)], sems[g])
        for g in range(_GROUPS)
    ]

    lane = lax.iota(jnp.int32, 16)
    for g in range(_GROUPS):
        handles[g].wait()
        rbase = lane * (_SEG + 1) + jnp.int32(g * _CHUNK)
        zero = jnp.zeros((16,), jnp.float32)

        def body(t, carry):
            accs, idx = list(carry[0]), carry[1]
            for u in range(16):
                accs[u % 4] = accs[u % 4] + plsc.load_gather(buf, [idx])
                idx = idx + 1
            return tuple(accs), idx

        accs, _ = lax.fori_loop(0, (_SEG - 16) // 16, body,
                                ((zero,) * 4, rbase))
        accs = list(accs)
        for i in range(_SEG - 16, _SEG):
            offs = jnp.full((16,), i, jnp.int32) + lane
            wrapped = jnp.where(offs >= _SEG, offs - _SEG, offs)
            accs[i % 4] = accs[i % 4] + plsc.load_gather(
                buf, [lane * _SEG + wrapped + jnp.int32(g * _CHUNK)])
        acc_ref[pl.ds(g * 16, 16)] = (accs[0] + accs[1]) + (accs[2] + accs[3])

    pltpu.sync_copy(acc_ref, out_hbm.at[row, pl.ds(half * _SEGS_PER_W,
                                                   _SEGS_PER_W)])


_sc_pool = functools.partial(
    pl.kernel,
    out_type=jax.ShapeDtypeStruct((_N, _J), jnp.float32),
    mesh=plsc.VectorSubcoreMesh(core_axis_name="c", subcore_axis_name="s",
                                num_cores=_NC, num_subcores=_NS),
    compiler_params=pltpu.CompilerParams(
        needs_layout_passes=False,
        disable_bounds_checks=True,
        disable_semaphore_checks=True,
    ),
    scratch_types=[
        pltpu.VMEM((_ELEMS_PER_W,), jnp.float32),
        pltpu.VMEM((_SEGS_PER_W,), jnp.float32),
        pltpu.SemaphoreType.DMA,
        pltpu.SemaphoreType.DMA,
        pltpu.SemaphoreType.DMA,
        pltpu.SemaphoreType.DMA,
    ],
)(_sc_pool_body)


def _fold_body(v0, g0, b0, v1, g1, b1, v2, g2, b2, v3, g3, b3,
               fcv, fcg, fcb, mt_ref, bias_ref):
    hp = lax.Precision.HIGHEST

    def wn(v, g, axis):
        n = jnp.sqrt(jnp.sum(v * v, axis=axis, keepdims=True) + 1e-12)
        return v * (g / n)

    W0 = wn(v0[...], g0[...], 0)
    W1 = wn(v1[...], g1[...], 0)
    W2 = wn(v2[...], g2[...], 0)
    W3 = wn(v3[...], g3[...], 0)

    a1 = jnp.dot(W0, W1, precision=hp)
    c1 = 4.0 * jnp.dot(b0[...], W1, precision=hp) + b1[...]
    a2 = a1 + jnp.dot(a1, W2, precision=hp)
    c2 = c1 + jnp.dot(c1, W2, precision=hp) + b2[...]
    a3 = jnp.dot(a2, W3, precision=hp)
    c3 = 128.0 * jnp.dot(c2, W3, precision=hp) + b3[...]

    fcw = wn(fcv[...], fcg[...], 1)
    fcw3 = fcw.reshape(_K, _J, 128)
    Mt = jnp.sum(fcw3 * a3[0][None, None, :], axis=-1)
    Mc = jnp.sum(fcw3 * c3[0][None, None, :], axis=-1)

    mt_ref[...] = Mt
    ones = jnp.ones((1, _J), jnp.float32)
    bias_ref[...] = lax.dot_general(ones, Mc, _DN, precision=hp) + fcb[...]


def _final_body(s_ref, mt_ref, bias_ref, out_ref):
    out_ref[...] = (
        lax.dot_general(s_ref[...], mt_ref[...], _DN,
                        precision=lax.Precision.HIGHEST)
        + bias_ref[...]
    )


def kernel(x, V0, g0, b0, V1, g1, b1, V2, g2, b2, V3, g3, b3, fcV, fcg, fcb):
    s = _sc_pool(x)

    mt, bias = pl.pallas_call(
        _fold_body,
        out_shape=[
            jax.ShapeDtypeStruct((_K, _J), jnp.float32),
            jax.ShapeDtypeStruct((1, _K), jnp.float32),
        ],
    )(
        V0, g0.reshape(1, -1), b0.reshape(1, -1),
        V1, g1.reshape(1, -1), b1.reshape(1, -1),
        V2, g2.reshape(1, -1), b2.reshape(1, -1),
        V3, g3.reshape(1, -1), b3.reshape(1, -1),
        fcV, fcg.reshape(-1, 1), fcb.reshape(1, -1),
    )

    return pl.pallas_call(
        _final_body,
        out_shape=jax.ShapeDtypeStruct((_N, _K), jnp.float32),
    )(s, mt, bias)


# speedup vs baseline: 1.0599x; 1.0599x over previous
import functools

import jax
import jax.numpy as jnp
from jax import lax
from jax.experimental import pallas as pl
from jax.experimental.pallas import tpu as pltpu
from jax.experimental.pallas import tpu_sc as plsc

_N = 16
_J = 128
_SEG = 1024
_K = 20

_NC, _NS = 2, 16
_NW = _NC * _NS
_SEGS_PER_W = (_N * _J) // _NW
_ELEMS_PER_W = _SEGS_PER_W * _SEG
_GROUPS = _SEGS_PER_W // 16
_CHUNK = 16 * _SEG

_DN = (((1,), (1,)), ((), ()))


def _sc_pool_body(x_hbm, out_hbm, buf, acc_ref, sem):
    wid = lax.axis_index("s") * _NC + lax.axis_index("c")
    row = wid // 2
    half = wid % 2
    col = half * _ELEMS_PER_W

    pltpu.async_copy(x_hbm.at[row, pl.ds(col, _ELEMS_PER_W)], buf,
                     sem).wait()

    lane = lax.iota(jnp.int32, 16)

    def group_body(g, _):
        gbase = g * _CHUNK
        rbase = lane * (_SEG + 1) + gbase
        zero = jnp.zeros((16,), jnp.float32)

        def body(t, carry):
            accs, idx = list(carry[0]), carry[1]
            for u in range(16):
                accs[u % 4] = accs[u % 4] + plsc.load_gather(buf, [idx])
                idx = idx + 1
            return tuple(accs), idx

        accs, _ = lax.fori_loop(0, (_SEG - 16) // 16, body,
                                ((zero,) * 4, rbase))
        accs = list(accs)
        for i in range(_SEG - 16, _SEG):
            offs = jnp.full((16,), i, jnp.int32) + lane
            wrapped = jnp.where(offs >= _SEG, offs - _SEG, offs)
            accs[i % 4] = accs[i % 4] + plsc.load_gather(
                buf, [lane * _SEG + wrapped + gbase])
        acc_ref[pl.ds(g * 16, 16)] = (accs[0] + accs[1]) + (accs[2] + accs[3])
        return 0

    lax.fori_loop(0, _GROUPS, group_body, 0)

    pltpu.sync_copy(acc_ref, out_hbm.at[row, pl.ds(half * _SEGS_PER_W,
                                                   _SEGS_PER_W)])


_sc_pool = functools.partial(
    pl.kernel,
    out_type=jax.ShapeDtypeStruct((_N, _J), jnp.float32),
    mesh=plsc.VectorSubcoreMesh(core_axis_name="c", subcore_axis_name="s",
                                num_cores=_NC, num_subcores=_NS),
    compiler_params=pltpu.CompilerParams(
        needs_layout_passes=False,
        disable_bounds_checks=True,
        disable_semaphore_checks=True,
    ),
    scratch_types=[
        pltpu.VMEM((_ELEMS_PER_W,), jnp.float32),
        pltpu.VMEM((_SEGS_PER_W,), jnp.float32),
        pltpu.SemaphoreType.DMA,
    ],
)(_sc_pool_body)


def _fold_body(v0, g0, b0, v1, g1, b1, v2, g2, b2, v3, g3, b3,
               fcv, fcg, fcb, mt_ref, bias_ref):
    hp = lax.Precision.HIGHEST

    def wn(v, g, axis):
        n = jnp.sqrt(jnp.sum(v * v, axis=axis, keepdims=True) + 1e-12)
        return v * (g / n)

    W0 = wn(v0[...], g0[...], 0)
    W1 = wn(v1[...], g1[...], 0)
    W2 = wn(v2[...], g2[...], 0)
    W3 = wn(v3[...], g3[...], 0)

    a1 = jnp.dot(W0, W1, precision=hp)
    c1 = 4.0 * jnp.dot(b0[...], W1, precision=hp) + b1[...]
    a2 = a1 + jnp.dot(a1, W2, precision=hp)
    c2 = c1 + jnp.dot(c1, W2, precision=hp) + b2[...]
    a3 = jnp.dot(a2, W3, precision=hp)
    c3 = 128.0 * jnp.dot(c2, W3, precision=hp) + b3[...]

    fcw = wn(fcv[...], fcg[...], 1)
    fcw3 = fcw.reshape(_K, _J, 128)
    Mt = jnp.sum(fcw3 * a3[0][None, None, :], axis=-1)
    Mc = jnp.sum(fcw3 * c3[0][None, None, :], axis=-1)

    mt_ref[...] = Mt
    ones = jnp.ones((1, _J), jnp.float32)
    bias_ref[...] = lax.dot_general(ones, Mc, _DN, precision=hp) + fcb[...]


def _final_body(s_ref, mt_ref, bias_ref, out_ref):
    out_ref[...] = (
        lax.dot_general(s_ref[...], mt_ref[...], _DN,
                        precision=lax.Precision.HIGHEST)
        + bias_ref[...]
    )


def kernel(x, V0, g0, b0, V1, g1, b1, V2, g2, b2, V3, g3, b3, fcV, fcg, fcb):
    s = _sc_pool(x)

    mt, bias = pl.pallas_call(
        _fold_body,
        out_shape=[
            jax.ShapeDtypeStruct((_K, _J), jnp.float32),
            jax.ShapeDtypeStruct((1, _K), jnp.float32),
        ],
    )(
        V0, g0.reshape(1, -1), b0.reshape(1, -1),
        V1, g1.reshape(1, -1), b1.reshape(1, -1),
        V2, g2.reshape(1, -1), b2.reshape(1, -1),
        V3, g3.reshape(1, -1), b3.reshape(1, -1),
        fcV, fcg.reshape(-1, 1), fcb.reshape(1, -1),
    )

    return pl.pallas_call(
        _final_body,
        out_shape=jax.ShapeDtypeStruct((_N, _K), jnp.float32),
    )(s, mt, bias)


# speedup vs baseline: 1.1426x; 1.0780x over previous
import functools

import jax
import jax.numpy as jnp
from jax import lax
from jax.experimental import pallas as pl
from jax.experimental.pallas import tpu as pltpu
from jax.experimental.pallas import tpu_sc as plsc

_N = 16
_J = 128
_SEG = 1024
_K = 20
_COLS = _J * _SEG

_NC, _NS = 2, 16
_NW = _NC * _NS

_RSC = 8
_RTC = _N - _RSC
_WPR = _NW // _RSC
_SEGS_W = _J // _WPR
_ELEMS_W = _SEGS_W * _SEG
_GROUPS = _SEGS_W // 16
_CHUNK = 16 * _SEG

_DN = (((1,), (1,)), ((), ()))


def _sc_pool_body(x_hbm, out_hbm, buf, acc_ref, sem):
    wid = lax.axis_index("s") * _NC + lax.axis_index("c")
    row = wid // _WPR
    part = wid % _WPR
    pltpu.async_copy(x_hbm.at[row, pl.ds(part * _ELEMS_W, _ELEMS_W)], buf,
                     sem).wait()

    lane = lax.iota(jnp.int32, 16)

    def group_body(g, _):
        gbase = g * _CHUNK
        rbase = lane * (_SEG + 1) + gbase
        zero = jnp.zeros((16,), jnp.float32)

        def body(t, carry):
            accs, idx = list(carry[0]), carry[1]
            for u in range(16):
                accs[u % 4] = accs[u % 4] + plsc.load_gather(buf, [idx])
                idx = idx + 1
            return tuple(accs), idx

        accs, _ = lax.fori_loop(0, (_SEG - 16) // 16, body,
                                ((zero,) * 4, rbase))
        accs = list(accs)
        for i in range(_SEG - 16, _SEG):
            offs = jnp.full((16,), i, jnp.int32) + lane
            wrapped = jnp.where(offs >= _SEG, offs - _SEG, offs)
            accs[i % 4] = accs[i % 4] + plsc.load_gather(
                buf, [lane * _SEG + wrapped + gbase])
        acc_ref[pl.ds(g * 16, 16)] = (accs[0] + accs[1]) + (accs[2] + accs[3])
        return 0

    lax.fori_loop(0, _GROUPS, group_body, 0)

    pltpu.sync_copy(acc_ref, out_hbm.at[row, pl.ds(part * _SEGS_W, _SEGS_W)])


_sc_pool = functools.partial(
    pl.kernel,
    out_type=jax.ShapeDtypeStruct((_RSC, _J), jnp.float32),
    mesh=plsc.VectorSubcoreMesh(core_axis_name="c", subcore_axis_name="s",
                                num_cores=_NC, num_subcores=_NS),
    compiler_params=pltpu.CompilerParams(
        needs_layout_passes=False,
        disable_bounds_checks=True,
        disable_semaphore_checks=True,
    ),
    scratch_types=[
        pltpu.VMEM((_ELEMS_W,), jnp.float32),
        pltpu.VMEM((_SEGS_W,), jnp.float32),
        pltpu.SemaphoreType.DMA,
    ],
)(_sc_pool_body)

_GRID_TC = 2
_CBLK = _COLS // _GRID_TC
_JBLK = _J // _GRID_TC


def _fold_body(x_ref, v0, g0, b0, v1, g1, b1, v2, g2, b2, v3, g3, b3,
               fcv, fcg, fcb, stc_ref, mt_ref, bias_ref):
    hp = lax.Precision.HIGHEST
    i = pl.program_id(0)

    xb = x_ref[...].reshape(_RTC, _JBLK, _SEG)
    stc_ref[...] = jnp.sum(xb, axis=-1)[None]

    @pl.when(i == _GRID_TC - 1)
    def _():
        def wn(v, g, axis):
            n = jnp.sqrt(jnp.sum(v * v, axis=axis, keepdims=True) + 1e-12)
            return v * (g / n)

        W0 = wn(v0[...], g0[...], 0)
        W1 = wn(v1[...], g1[...], 0)
        W2 = wn(v2[...], g2[...], 0)
        W3 = wn(v3[...], g3[...], 0)

        a1 = jnp.dot(W0, W1, precision=hp)
        c1 = 4.0 * jnp.dot(b0[...], W1, precision=hp) + b1[...]
        a2 = a1 + jnp.dot(a1, W2, precision=hp)
        c2 = c1 + jnp.dot(c1, W2, precision=hp) + b2[...]
        a3 = jnp.dot(a2, W3, precision=hp)
        c3 = 128.0 * jnp.dot(c2, W3, precision=hp) + b3[...]

        fcw = wn(fcv[...], fcg[...], 1)
        fcw3 = fcw.reshape(_K, _J, 128)
        Mt = jnp.sum(fcw3 * a3[0][None, None, :], axis=-1)
        Mc = jnp.sum(fcw3 * c3[0][None, None, :], axis=-1)

        mt_ref[...] = Mt
        ones = jnp.ones((1, _J), jnp.float32)
        bias_ref[...] = lax.dot_general(ones, Mc, _DN, precision=hp) + fcb[...]


def _final_body(ssc_ref, stc_ref, mt_ref, bias_ref, out_ref):
    s_tc = jnp.concatenate([stc_ref[0], stc_ref[1]], axis=1)
    s = jnp.concatenate([ssc_ref[...], s_tc], axis=0)
    out_ref[...] = (
        lax.dot_general(s, mt_ref[...], _DN, precision=lax.Precision.HIGHEST)
        + bias_ref[...]
    )


def _full(shape):
    return pl.BlockSpec(shape, lambda i: (0,) * len(shape))


def kernel(x, V0, g0, b0, V1, g1, b1, V2, g2, b2, V3, g3, b3, fcV, fcg, fcb):
    s_sc = _sc_pool(x)

    wargs = (
        V0, g0.reshape(1, -1), b0.reshape(1, -1),
        V1, g1.reshape(1, -1), b1.reshape(1, -1),
        V2, g2.reshape(1, -1), b2.reshape(1, -1),
        V3, g3.reshape(1, -1), b3.reshape(1, -1),
        fcV, fcg.reshape(-1, 1), fcb.reshape(1, -1),
    )
    s_tc, mt, bias = pl.pallas_call(
        _fold_body,
        grid=(_GRID_TC,),
        in_specs=[pl.BlockSpec((_RTC, _CBLK), lambda i: (1, i))]
        + [_full(w.shape) for w in wargs],
        out_specs=[
            pl.BlockSpec((1, _RTC, _JBLK), lambda i: (i, 0, 0)),
            _full((_K, _J)),
            _full((1, _K)),
        ],
        out_shape=[
            jax.ShapeDtypeStruct((_GRID_TC, _RTC, _JBLK), jnp.float32),
            jax.ShapeDtypeStruct((_K, _J), jnp.float32),
            jax.ShapeDtypeStruct((1, _K), jnp.float32),
        ],
    )(x, *wargs)

    return pl.pallas_call(
        _final_body,
        out_shape=jax.ShapeDtypeStruct((_N, _K), jnp.float32),
    )(s_sc, s_tc, mt, bias)
